# baseline (speedup 1.0000x reference)
import functools

import jax
import jax.numpy as jnp
from jax import lax
from jax.experimental import pallas as pl
from jax.experimental.pallas import tpu as pltpu
from jax.experimental.pallas import tpu_sc as plsc

N_CLS = 1000
PROMPT_LEN = 16
EMBED_DIM = 768
CTX_LEN = 77
SUF_LEN = CTX_LEN - 1 - PROMPT_LEN

_NC = 2
_NS = 16
_NW = _NC * _NS
_PER_W = (N_CLS + _NW - 1) // _NW


def _sc_body(pre_hbm, prm_hbm, suf_hbm, out_hbm, buf):
    wid = lax.axis_index("s") * _NC + lax.axis_index("c")

    def body(k, _):
        c = wid + k * _NW

        @pl.when(c < N_CLS)
        def _do():
            pltpu.sync_copy(pre_hbm.at[c], buf.at[pl.ds(0, 1)])
            pltpu.sync_copy(prm_hbm.at[c], buf.at[pl.ds(1, PROMPT_LEN)])
            pltpu.sync_copy(suf_hbm.at[c], buf.at[pl.ds(1 + PROMPT_LEN, SUF_LEN)])
            pltpu.sync_copy(buf, out_hbm.at[c])

        return _

    lax.fori_loop(0, _PER_W, body, None)


def kernel(new_prompt_tokens, token_prefix, token_suffix, tokenized_prompts):
    sc_call = pl.kernel(
        _sc_body,
        out_type=jax.ShapeDtypeStruct((N_CLS, CTX_LEN, EMBED_DIM), jnp.float32),
        mesh=plsc.VectorSubcoreMesh(core_axis_name="c", subcore_axis_name="s"),
        scratch_types=[
            pltpu.VMEM((CTX_LEN, EMBED_DIM), jnp.float32),
        ],
        compiler_params=pltpu.CompilerParams(use_tc_tiling_on_sc=False),
    )
    prompts = sc_call(token_prefix, new_prompt_tokens, token_suffix)
    return (tokenized_prompts, prompts)


# speedup vs baseline: 5.5904x; 5.5904x over previous
import jax
import jax.numpy as jnp
from jax import lax
from jax.experimental import pallas as pl
from jax.experimental.pallas import tpu as pltpu

N_CLS = 1000
PROMPT_LEN = 16
EMBED_DIM = 768
CTX_LEN = 77
SUF_LEN = CTX_LEN - 1 - PROMPT_LEN

C = 10
G = 4
NSTEP = N_CLS // C
NITER = NSTEP // G
NBUF = 2 * G


def _body(pre_hbm, prm_hbm, suf_hbm, out_hbm,
          pre_v, prm_v, suf_v, out_v,
          pre_s, prm_s, suf_s, out_s):
    i = pl.program_id(0)

    def in_copies(step):
        slot = lax.rem(step, NBUF)
        c0 = step * C
        return (
            pltpu.make_async_copy(pre_hbm.at[pl.ds(c0, C)], pre_v.at[slot], pre_s.at[slot]),
            pltpu.make_async_copy(prm_hbm.at[pl.ds(c0, C)], prm_v.at[slot], prm_s.at[slot]),
            pltpu.make_async_copy(suf_hbm.at[pl.ds(c0, C)], suf_v.at[slot], suf_s.at[slot]),
        )

    def start_in(step, g):
        pre_c, prm_c, suf_c = in_copies(step)
        pre_c.start(priority=g % 2)
        prm_c.start(priority=g % 2)
        suf_c.start(priority=g % 2)

    def out_copy(step):
        slot = lax.rem(step, NBUF)
        c0 = step * C
        return pltpu.make_async_copy(out_v.at[slot], out_hbm.at[pl.ds(c0, C)], out_s.at[slot])

    @pl.when(i == 0)
    def _prologue():
        for g in range(G):
            start_in(g, g)

    @pl.when(i + 1 < NITER)
    def _next_in():
        for g in range(G):
            start_in((i + 1) * G + g, g)

    for g in range(G):
        step = i * G + g
        pre_c, prm_c, suf_c = in_copies(step)
        pre_c.wait()
        prm_c.wait()
        suf_c.wait()

        @pl.when(i >= 2)
        def _wait_prev_out():
            out_copy(step - NBUF).wait()

        slot = lax.rem(step, NBUF)
        out_v[slot] = jnp.concatenate(
            [pre_v[slot], prm_v[slot], suf_v[slot]], axis=1)
        out_copy(step).start(priority=g % 2)

    @pl.when(i == NITER - 1)
    def _drain():
        for j in range(NBUF):
            out_copy(NSTEP - 1 - j).wait()


def kernel(new_prompt_tokens, token_prefix, token_suffix, tokenized_prompts):
    prompts = pl.pallas_call(
        _body,
        grid=(NITER,),
        in_specs=[
            pl.BlockSpec(memory_space=pl.ANY),
            pl.BlockSpec(memory_space=pl.ANY),
            pl.BlockSpec(memory_space=pl.ANY),
        ],
        out_specs=pl.BlockSpec(memory_space=pl.ANY),
        out_shape=jax.ShapeDtypeStruct((N_CLS, CTX_LEN, EMBED_DIM), jnp.float32),
        scratch_shapes=[
            pltpu.VMEM((NBUF, C, 1, EMBED_DIM), jnp.float32),
            pltpu.VMEM((NBUF, C, PROMPT_LEN, EMBED_DIM), jnp.float32),
            pltpu.VMEM((NBUF, C, SUF_LEN, EMBED_DIM), jnp.float32),
            pltpu.VMEM((NBUF, C, CTX_LEN, EMBED_DIM), jnp.float32),
            pltpu.SemaphoreType.DMA((NBUF,)),
            pltpu.SemaphoreType.DMA((NBUF,)),
            pltpu.SemaphoreType.DMA((NBUF,)),
            pltpu.SemaphoreType.DMA((NBUF,)),
        ],
        compiler_params=pltpu.CompilerParams(
            dimension_semantics=("arbitrary",),
        ),
    )(token_prefix, new_prompt_tokens, token_suffix)
    return (tokenized_prompts, prompts)
